# baseline (speedup 1.0000x reference)
import functools

import jax
import jax.numpy as jnp
from jax import lax
from jax.experimental import pallas as pl
from jax.experimental.pallas import tpu as pltpu
from jax.experimental.pallas import tpu_sc as plsc

_F32 = jnp.float32


def _wk_body(ew_ref, et_ref, sc_ref, w1_ref, ex1_ref, w2_ref, ex2_ref):
    ew = ew_ref[...]
    et = et_ref[...]
    e_count = ew.size
    mu = jnp.sum(ew) / e_count
    var = jnp.sum(ew * ew) / e_count - mu * mu
    inv = lax.rsqrt(var + 1e-5)
    etmax = jnp.max(et)
    etmin = jnp.min(et)
    for l, (w_ref, ex_ref) in enumerate([(w1_ref, ex1_ref), (w2_ref, ex2_ref)]):
        ge = sc_ref[0, 4 * l + 0]
        be = sc_ref[0, 4 * l + 1]
        delta = sc_ref[0, 4 * l + 2]
        zw = (ew - mu) * inv * ge + be
        cmax = jnp.maximum(delta * etmax, delta * etmin)
        ex = jnp.exp(delta * et - cmax)
        ex_ref[...] = ex
        w_ref[...] = zw * ex


def _edge_weights(ew, et, ge1, be1, d1, ge2, be2, d2, interpret=False):
    e_count = ew.shape[0]
    r = 128
    m = e_count // r
    ew2 = ew.reshape(m, r)
    et2 = et.reshape(m, r)
    sc = jnp.stack([ge1, be1, d1, jnp.zeros((), _F32),
                    ge2, be2, d2, jnp.zeros((), _F32)]).reshape(1, 8).astype(_F32)
    outs = pl.pallas_call(
        _wk_body,
        out_shape=[jax.ShapeDtypeStruct((m, r), _F32)] * 4,
        in_specs=[
            pl.BlockSpec((m, r), lambda: (0, 0)),
            pl.BlockSpec((m, r), lambda: (0, 0)),
            pl.BlockSpec(memory_space=pltpu.SMEM),
        ],
        out_specs=[pl.BlockSpec((m, r), lambda: (0, 0))] * 4,
        interpret=interpret,
    )(ew2, et2, sc)
    return [o.reshape(e_count) for o in outs]


def _edge_call(src_r, dst_r, w_r, ex_r, x, zeros, nc, ns, interpret=False):
    nw, sgs, sub, k = src_r.shape
    n, d = x.shape
    cp = -(-n // ns)
    cp += (-cp) % 8
    last = n - (ns - 1) * cp
    assert last > 0 and last % 8 == 0
    assert sub >= 2
    mesh = plsc.VectorSubcoreMesh(core_axis_name="c", subcore_axis_name="s",
                                  num_cores=nc, num_subcores=ns)

    @functools.partial(
        pl.kernel,
        out_type=[jax.ShapeDtypeStruct((nc, n, d), _F32),
                  jax.ShapeDtypeStruct((nc, 1, n), _F32)],
        mesh=mesh,
        scratch_types=[
            pltpu.VMEM((sub, k), jnp.int32),
            pltpu.VMEM((sub, k), jnp.int32),
            pltpu.VMEM((sub, k), _F32),
            pltpu.VMEM((sub, k), _F32),
            pltpu.VMEM((k, d), _F32),
            pltpu.VMEM((k, d), _F32),
            pltpu.VMEM((640,), _F32),
            pltpu.VMEM_SHARED((n, d), _F32),
            pltpu.VMEM_SHARED((n,), _F32),
            pltpu.SemaphoreType.DMA,
            pltpu.SemaphoreType.DMA,
            pltpu.SemaphoreType.DMA,
            pltpu.SemaphoreType.DMA,
            pltpu.SemaphoreType.DMA,
        ],
        compiler_params=pltpu.CompilerParams(needs_layout_passes=False),
        interpret=interpret,
    )
    def edge_k(src_ref, dst_ref, w_ref, ex_ref, x_ref, z_ref, raw_out,
               den_out, srcv, dstv, wv, exv, rows0, rows1, zden, racc, dacc,
               semg0, semg1, semv0, semv1, semd):
        c = lax.axis_index("c")
        s = lax.axis_index("s")
        tid = c * ns + s
        zeros16 = jnp.zeros((16,), _F32)

        def zd_body(i, carry):
            zden[pl.ds(i * 16, 16)] = zeros16
            return carry

        lax.fori_loop(0, 40, zd_body, 0)

        base_row = pl.multiple_of(s * cp, 8)

        @pl.when(s < ns - 1)
        def _():
            pltpu.sync_copy(z_ref.at[pl.ds(base_row, cp)],
                            racc.at[pl.ds(base_row, cp)])
            pltpu.sync_copy(zden.at[pl.ds(0, cp)],
                            dacc.at[pl.ds(base_row, cp)])

        @pl.when(s == ns - 1)
        def _():
            pltpu.sync_copy(z_ref.at[pl.ds(base_row, last)],
                            racc.at[pl.ds(base_row, last)])
            pltpu.sync_copy(zden.at[pl.ds(0, last)],
                            dacc.at[pl.ds(base_row, last)])

        plsc.subcore_barrier()

        gsem = [semg0, semg1]
        ssem = [semv0, semv1]
        rbuf = [rows0, rows1]

        def gather_start(g, b):
            pltpu.async_copy(x_ref.at[srcv.at[g]], rbuf[b], gsem[b])

        def gather_wait(g, b):
            pltpu.make_async_copy(x_ref.at[srcv.at[g]], rbuf[b],
                                  gsem[b]).wait()

        def scatter_start(g, b):
            pltpu.async_copy(rbuf[b], racc.at[dstv.at[g]], ssem[b], add=True)

        def scatter_wait(g, b):
            pltpu.make_async_copy(rbuf[b], racc.at[dstv.at[g]],
                                  ssem[b]).wait()

        def den_start(g):
            pltpu.async_copy(exv.at[g], dacc.at[dstv.at[g]], semd, add=True)

        def den_wait(g):
            pltpu.make_async_copy(exv.at[g], dacc.at[dstv.at[g]],
                                  semd).wait()

        def work(g, b):
            gather_wait(g, b)

            @pl.when(g >= 1)
            def _():
                scatter_wait(g - 1, 1 - b)
                den_wait(g - 1)

            @pl.when(g + 1 < sub)
            def _():
                gather_start(g + 1, 1 - b)

            den_start(g)
            rows = rbuf[b]

            def scale(t, c2):
                w16 = wv[g, pl.ds(t * 16, 16)]
                base = t * 16
                for ll in range(16):
                    wk = w16[ll]
                    for j in range(d // 16):
                        sl = pl.ds(j * 16, 16)
                        rows[base + ll, sl] = rows[base + ll, sl] * wk
                return c2

            lax.fori_loop(0, k // 16, scale, 0)
            scatter_start(g, b)

        def stage(sg, carry):
            pltpu.sync_copy(src_ref.at[tid, sg], srcv)
            pltpu.sync_copy(dst_ref.at[tid, sg], dstv)
            pltpu.sync_copy(w_ref.at[tid, sg], wv)
            pltpu.sync_copy(ex_ref.at[tid, sg], exv)

            gather_start(0, 0)

            def chunk(g, carry2):
                @pl.when(g % 2 == 0)
                def _():
                    work(g, 0)

                @pl.when(g % 2 == 1)
                def _():
                    work(g, 1)

                return carry2

            lax.fori_loop(0, sub, chunk, 0)
            scatter_wait(sub - 1, (sub - 1) % 2)
            den_wait(sub - 1)
            return carry

        lax.fori_loop(0, sgs, stage, 0)
        plsc.subcore_barrier()

        @pl.when(s < ns - 1)
        def _():
            pltpu.sync_copy(racc.at[pl.ds(base_row, cp)],
                            raw_out.at[c, pl.ds(base_row, cp)])

        @pl.when(s == ns - 1)
        def _():
            pltpu.sync_copy(racc.at[pl.ds(base_row, last)],
                            raw_out.at[c, pl.ds(base_row, last)])

        @pl.when(s == 0)
        def _():
            pltpu.sync_copy(dacc, den_out.at[c, 0])

    return edge_k(src_r, dst_r, w_r, ex_r, x, zeros)


def _dotT(a, b):
    return lax.dot_general(a, b, (((1,), (1,)), ((), ())),
                           preferred_element_type=_F32)


def _u_body(x_ref, raw_ref, den_ref, ws_ref, bs_ref, wh_ref, bh_ref,
            h_ref, st_ref):
    i = pl.program_id(0)
    blk = x_ref.shape[0]
    den = jnp.sum(den_ref[...], axis=1)
    raws = raw_ref[0] + raw_ref[1]
    safe = jnp.maximum(den, jnp.float32(1e-30))
    aggr = jnp.where(den[:, None] > 0, raws / safe[:, None], jnp.float32(0))
    h = (_dotT(x_ref[...], ws_ref[...]) + bs_ref[0][None, :]
         + _dotT(aggr, wh_ref[...]) + bh_ref[0][None, :])
    h_ref[...] = h
    cs = jnp.sum(h, axis=0)
    cq = jnp.sum(h * h, axis=0)
    acc = jnp.concatenate(
        [cs[None], cq[None], jnp.zeros((6, cs.shape[0]), _F32)], axis=0)

    @pl.when(i == 0)
    def _():
        st_ref[...] = acc

    @pl.when(i != 0)
    def _():
        st_ref[...] = st_ref[...] + acc


def _u_call(x, raw, den, ws, bs, wh, bh, interpret=False):
    n, d = x.shape
    nw = den.shape[1]
    blk = 2000
    grid = n // blk
    return pl.pallas_call(
        _u_body,
        grid=(grid,),
        out_shape=[jax.ShapeDtypeStruct((n, d), _F32),
                   jax.ShapeDtypeStruct((8, d), _F32)],
        in_specs=[
            pl.BlockSpec((blk, d), lambda i: (i, 0)),
            pl.BlockSpec((2, blk, d), lambda i: (0, i, 0)),
            pl.BlockSpec((blk, nw), lambda i: (i, 0)),
            pl.BlockSpec((d, d), lambda i: (0, 0)),
            pl.BlockSpec((1, d), lambda i: (0, 0)),
            pl.BlockSpec((d, d), lambda i: (0, 0)),
            pl.BlockSpec((1, d), lambda i: (0, 0)),
        ],
        out_specs=[
            pl.BlockSpec((blk, d), lambda i: (i, 0)),
            pl.BlockSpec((8, d), lambda i: (0, 0)),
        ],
        interpret=interpret,
    )(x, raw, den, ws, bs.reshape(1, d), wh, bh.reshape(1, d))


def _v_body_mk(n_total, add_residual):
    if add_residual:
        def body(h_ref, st_ref, gf_ref, bf_ref, wf_ref, bfc_ref, x_ref,
                 out_ref):
            mu = st_ref[0] / n_total
            var = st_ref[1] / n_total - mu * mu
            scale = lax.rsqrt(var + 1e-5) * gf_ref[0]
            hn = (h_ref[...] - mu[None, :]) * scale[None, :] + bf_ref[0][None, :]
            o = _dotT(jnp.maximum(hn, 0), wf_ref[...]) + bfc_ref[0][None, :]
            out_ref[...] = o + x_ref[...]
    else:
        def body(h_ref, st_ref, gf_ref, bf_ref, wf_ref, bfc_ref, out_ref):
            mu = st_ref[0] / n_total
            var = st_ref[1] / n_total - mu * mu
            scale = lax.rsqrt(var + 1e-5) * gf_ref[0]
            hn = (h_ref[...] - mu[None, :]) * scale[None, :] + bf_ref[0][None, :]
            o = _dotT(jnp.maximum(hn, 0), wf_ref[...]) + bfc_ref[0][None, :]
            out_ref[...] = o
    return body


def _v_call(h, st, gf, bf, wf, bfc, x_add=None, interpret=False):
    n, d = h.shape
    blk = 2000
    grid = n // blk
    in_specs = [
        pl.BlockSpec((blk, d), lambda i: (i, 0)),
        pl.BlockSpec((8, d), lambda i: (0, 0)),
        pl.BlockSpec((1, d), lambda i: (0, 0)),
        pl.BlockSpec((1, d), lambda i: (0, 0)),
        pl.BlockSpec((d, d), lambda i: (0, 0)),
        pl.BlockSpec((1, d), lambda i: (0, 0)),
    ]
    args = [h, st, gf.reshape(1, d), bf.reshape(1, d), wf, bfc.reshape(1, d)]
    if x_add is not None:
        in_specs.append(pl.BlockSpec((blk, d), lambda i: (i, 0)))
        args.append(x_add)
    return pl.pallas_call(
        _v_body_mk(n, x_add is not None),
        grid=(grid,),
        out_shape=jax.ShapeDtypeStruct((n, d), _F32),
        in_specs=in_specs,
        out_specs=pl.BlockSpec((blk, d), lambda i: (i, 0)),
        interpret=interpret,
    )(*args)


def kernel(x, edge_index, edge_weights, edge_times, node_time,
           delta1, ge1, be1, Ws1, bs1, Wh1, bh1, gf1, bf1, Wf1, bfc1,
           delta2, ge2, be2, Ws2, bs2, Wh2, bh2, gf2, bf2, Wf2, bfc2):
    n, d = x.shape
    e_count = edge_index.shape[1]
    src = edge_index[0]
    dst = edge_index[1]

    info = plsc.get_sparse_core_info()
    nc, ns = info.num_cores, info.num_subcores
    nw = nc * ns
    k = 128
    ch = -(-e_count // (nw * k))
    def _subfor(chv):
        for cand in (25, 20, 16, 10, 8, 5, 4):
            if chv % cand == 0:
                return cand
        return None
    while _subfor(ch) is None:
        ch += 1
    sub = _subfor(ch)
    sgs = ch // sub
    shp = (nw, sgs, sub, k)
    e_pad = nw * ch * k - e_count

    w1, ex1, w2, ex2 = _edge_weights(edge_weights, edge_times,
                                     ge1, be1, delta1, ge2, be2, delta2)

    zpad_i = jnp.zeros((e_pad,), jnp.int32)
    zpad_f = jnp.zeros((e_pad,), _F32)
    src = jnp.concatenate([src, zpad_i])
    dst = jnp.concatenate([dst, zpad_i])
    w1 = jnp.concatenate([w1, zpad_f])
    ex1 = jnp.concatenate([ex1, zpad_f])
    w2 = jnp.concatenate([w2, zpad_f])
    ex2 = jnp.concatenate([ex2, zpad_f])

    src_r = src.reshape(shp)
    dst_r = dst.reshape(shp)

    zeros = jnp.zeros((n, d), _F32)
    raw1, den1 = _edge_call(src_r, dst_r, w1.reshape(shp),
                            ex1.reshape(shp), x, zeros, nc, ns)
    h1, st1 = _u_call(x, raw1, den1.reshape(nc, n).T, Ws1, bs1, Wh1, bh1)
    x2 = _v_call(h1, st1, gf1, bf1, Wf1, bfc1, x_add=x)

    raw2, den2 = _edge_call(src_r, dst_r, w2.reshape(shp),
                            ex2.reshape(shp), x2, zeros, nc, ns)
    h2, st2 = _u_call(x2, raw2, den2.reshape(nc, n).T, Ws2, bs2, Wh2, bh2)
    out = _v_call(h2, st2, gf2, bf2, Wf2, bfc2, x_add=None)
    return out


# speedup vs baseline: 2.5510x; 2.5510x over previous
import functools

import jax
import jax.numpy as jnp
from jax import lax
from jax.experimental import pallas as pl
from jax.experimental.pallas import tpu as pltpu
from jax.experimental.pallas import tpu_sc as plsc

_F32 = jnp.float32


def _wk_body(ew_ref, et_ref, sc_ref, w1_ref, ex1_ref, w2_ref, ex2_ref):
    ew = ew_ref[...]
    et = et_ref[...]
    e_count = ew.size
    mu = jnp.sum(ew) / e_count
    var = jnp.sum(ew * ew) / e_count - mu * mu
    inv = lax.rsqrt(var + 1e-5)
    etmax = jnp.max(et)
    etmin = jnp.min(et)
    for l, (w_ref, ex_ref) in enumerate([(w1_ref, ex1_ref), (w2_ref, ex2_ref)]):
        ge = sc_ref[0, 4 * l + 0]
        be = sc_ref[0, 4 * l + 1]
        delta = sc_ref[0, 4 * l + 2]
        zw = (ew - mu) * inv * ge + be
        cmax = jnp.maximum(delta * etmax, delta * etmin)
        ex = jnp.exp(delta * et - cmax)
        ex_ref[...] = ex
        w_ref[...] = zw * ex


def _edge_weights(ew, et, ge1, be1, d1, ge2, be2, d2, interpret=False):
    e_count = ew.shape[0]
    r = 128
    m = e_count // r
    ew2 = ew.reshape(m, r)
    et2 = et.reshape(m, r)
    sc = jnp.stack([ge1, be1, d1, jnp.zeros((), _F32),
                    ge2, be2, d2, jnp.zeros((), _F32)]).reshape(1, 8).astype(_F32)
    outs = pl.pallas_call(
        _wk_body,
        out_shape=[jax.ShapeDtypeStruct((m, r), _F32)] * 4,
        in_specs=[
            pl.BlockSpec((m, r), lambda: (0, 0)),
            pl.BlockSpec((m, r), lambda: (0, 0)),
            pl.BlockSpec(memory_space=pltpu.SMEM),
        ],
        out_specs=[pl.BlockSpec((m, r), lambda: (0, 0))] * 4,
        interpret=interpret,
    )(ew2, et2, sc)
    return [o.reshape(e_count) for o in outs]


def _edge_call(src_r, dst_r, w_r, ex_r, x, nc, ns, interpret=False):
    nw, sgs, sub, k = src_r.shape
    n, d = x.shape
    cp = -(-n // ns)
    cp += (-cp) % 8
    last = n - (ns - 1) * cp
    assert last > 0 and last % 8 == 0
    assert sub >= 2
    zr = 32
    mesh = plsc.VectorSubcoreMesh(core_axis_name="c", subcore_axis_name="s",
                                  num_cores=nc, num_subcores=ns)

    @functools.partial(
        pl.kernel,
        out_type=[jax.ShapeDtypeStruct((nc, n, d), _F32),
                  jax.ShapeDtypeStruct((nc, 1, n), _F32)],
        mesh=mesh,
        scratch_types=[
            pltpu.VMEM((sub, k), jnp.int32),
            pltpu.VMEM((sub, k), jnp.int32),
            pltpu.VMEM((sub, k), _F32),
            pltpu.VMEM((sub, k), _F32),
            pltpu.VMEM((k, d), _F32),
            pltpu.VMEM((k, d), _F32),
            pltpu.VMEM((640,), _F32),
            pltpu.VMEM((zr, d), _F32),
            pltpu.VMEM_SHARED((n, d), _F32),
            pltpu.VMEM_SHARED((n,), _F32),
            pltpu.SemaphoreType.DMA,
            pltpu.SemaphoreType.DMA,
        ],
        compiler_params=pltpu.CompilerParams(needs_layout_passes=False),
        interpret=interpret,
    )
    def edge_k(src_ref, dst_ref, w_ref, ex_ref, x_ref, raw_out, den_out,
               srcv, dstv, wv, exv, rows0, rows1, zden, zbuf, racc, dacc,
               semg0, semg1):
        c = lax.axis_index("c")
        s = lax.axis_index("s")
        tid = c * ns + s
        zeros16 = jnp.zeros((16,), _F32)

        def zb_body(r_i, carry):
            for j in range(d // 16):
                zbuf[r_i, pl.ds(j * 16, 16)] = zeros16
            return carry

        lax.fori_loop(0, zr, zb_body, 0)

        def zd_body(i, carry):
            zden[pl.ds(i * 16, 16)] = zeros16
            return carry

        lax.fori_loop(0, 40, zd_body, 0)

        base_row = pl.multiple_of(s * cp, 8)

        def _zero_span(span):
            off = 0
            while off < span:
                cnt = min(zr, span - off)
                pltpu.sync_copy(
                    zbuf.at[pl.ds(0, cnt)],
                    racc.at[pl.ds(pl.multiple_of(base_row + off, 8), cnt)])
                off += cnt

        @pl.when(s < ns - 1)
        def _():
            _zero_span(cp)
            pltpu.sync_copy(zden.at[pl.ds(0, cp)],
                            dacc.at[pl.ds(base_row, cp)])

        @pl.when(s == ns - 1)
        def _():
            _zero_span(last)
            pltpu.sync_copy(zden.at[pl.ds(0, last)],
                            dacc.at[pl.ds(base_row, last)])

        plsc.subcore_barrier()

        gsem = [semg0, semg1]
        rbuf = [rows0, rows1]

        def gather_start(g, b):
            pltpu.async_copy(x_ref.at[srcv.at[g]], rbuf[b], gsem[b])

        def gather_wait(g, b):
            pltpu.make_async_copy(x_ref.at[srcv.at[g]], rbuf[b],
                                  gsem[b]).wait()

        def work(g, b):
            gather_wait(g, b)

            @pl.when(g + 1 < sub)
            def _():
                gather_start(g + 1, 1 - b)

            rows = rbuf[b]

            def scale(t, c2):
                w16 = wv[g, pl.ds(t * 16, 16)]
                base = t * 16
                for ll in range(16):
                    wk = w16[ll]
                    for j in range(d // 16):
                        sl = pl.ds(j * 16, 16)
                        rows[base + ll, sl] = rows[base + ll, sl] * wk
                return c2

            lax.fori_loop(0, k // 16, scale, 0)
            pltpu.sync_copy(exv.at[g], dacc.at[dstv.at[g]], add=True)
            pltpu.sync_copy(rows, racc.at[dstv.at[g]], add=True)

        def stage(sg, carry):
            pltpu.sync_copy(src_ref.at[tid, sg], srcv)
            pltpu.sync_copy(dst_ref.at[tid, sg], dstv)
            pltpu.sync_copy(w_ref.at[tid, sg], wv)
            pltpu.sync_copy(ex_ref.at[tid, sg], exv)

            gather_start(0, 0)

            def chunk(g, carry2):
                @pl.when(g % 2 == 0)
                def _():
                    work(g, 0)

                @pl.when(g % 2 == 1)
                def _():
                    work(g, 1)

                return carry2

            lax.fori_loop(0, sub, chunk, 0)
            return carry

        lax.fori_loop(0, sgs, stage, 0)
        plsc.subcore_barrier()

        @pl.when(s < ns - 1)
        def _():
            pltpu.sync_copy(racc.at[pl.ds(base_row, cp)],
                            raw_out.at[c, pl.ds(base_row, cp)])

        @pl.when(s == ns - 1)
        def _():
            pltpu.sync_copy(racc.at[pl.ds(base_row, last)],
                            raw_out.at[c, pl.ds(base_row, last)])

        @pl.when(s == 0)
        def _():
            pltpu.sync_copy(dacc, den_out.at[c, 0])

    return edge_k(src_r, dst_r, w_r, ex_r, x)


def _dotT(a, b):
    return lax.dot_general(a, b, (((1,), (1,)), ((), ())),
                           preferred_element_type=_F32)


def _u_body(x_ref, raw_ref, den_ref, ws_ref, bs_ref, wh_ref, bh_ref,
            h_ref, st_ref):
    i = pl.program_id(0)
    blk = x_ref.shape[0]
    den = jnp.sum(den_ref[...], axis=1)
    raws = raw_ref[0] + raw_ref[1]
    safe = jnp.maximum(den, jnp.float32(1e-30))
    aggr = jnp.where(den[:, None] > 0, raws / safe[:, None], jnp.float32(0))
    h = (_dotT(x_ref[...], ws_ref[...]) + bs_ref[0][None, :]
         + _dotT(aggr, wh_ref[...]) + bh_ref[0][None, :])
    h_ref[...] = h
    cs = jnp.sum(h, axis=0)
    cq = jnp.sum(h * h, axis=0)
    acc = jnp.concatenate(
        [cs[None], cq[None], jnp.zeros((6, cs.shape[0]), _F32)], axis=0)

    @pl.when(i == 0)
    def _():
        st_ref[...] = acc

    @pl.when(i != 0)
    def _():
        st_ref[...] = st_ref[...] + acc


def _u_call(x, raw, den, ws, bs, wh, bh, interpret=False):
    n, d = x.shape
    nw = den.shape[1]
    blk = 2000
    grid = n // blk
    return pl.pallas_call(
        _u_body,
        grid=(grid,),
        out_shape=[jax.ShapeDtypeStruct((n, d), _F32),
                   jax.ShapeDtypeStruct((8, d), _F32)],
        in_specs=[
            pl.BlockSpec((blk, d), lambda i: (i, 0)),
            pl.BlockSpec((2, blk, d), lambda i: (0, i, 0)),
            pl.BlockSpec((blk, nw), lambda i: (i, 0)),
            pl.BlockSpec((d, d), lambda i: (0, 0)),
            pl.BlockSpec((1, d), lambda i: (0, 0)),
            pl.BlockSpec((d, d), lambda i: (0, 0)),
            pl.BlockSpec((1, d), lambda i: (0, 0)),
        ],
        out_specs=[
            pl.BlockSpec((blk, d), lambda i: (i, 0)),
            pl.BlockSpec((8, d), lambda i: (0, 0)),
        ],
        interpret=interpret,
    )(x, raw, den, ws, bs.reshape(1, d), wh, bh.reshape(1, d))


def _v_body_mk(n_total, add_residual):
    if add_residual:
        def body(h_ref, st_ref, gf_ref, bf_ref, wf_ref, bfc_ref, x_ref,
                 out_ref):
            mu = st_ref[0] / n_total
            var = st_ref[1] / n_total - mu * mu
            scale = lax.rsqrt(var + 1e-5) * gf_ref[0]
            hn = (h_ref[...] - mu[None, :]) * scale[None, :] + bf_ref[0][None, :]
            o = _dotT(jnp.maximum(hn, 0), wf_ref[...]) + bfc_ref[0][None, :]
            out_ref[...] = o + x_ref[...]
    else:
        def body(h_ref, st_ref, gf_ref, bf_ref, wf_ref, bfc_ref, out_ref):
            mu = st_ref[0] / n_total
            var = st_ref[1] / n_total - mu * mu
            scale = lax.rsqrt(var + 1e-5) * gf_ref[0]
            hn = (h_ref[...] - mu[None, :]) * scale[None, :] + bf_ref[0][None, :]
            o = _dotT(jnp.maximum(hn, 0), wf_ref[...]) + bfc_ref[0][None, :]
            out_ref[...] = o
    return body


def _v_call(h, st, gf, bf, wf, bfc, x_add=None, interpret=False):
    n, d = h.shape
    blk = 2000
    grid = n // blk
    in_specs = [
        pl.BlockSpec((blk, d), lambda i: (i, 0)),
        pl.BlockSpec((8, d), lambda i: (0, 0)),
        pl.BlockSpec((1, d), lambda i: (0, 0)),
        pl.BlockSpec((1, d), lambda i: (0, 0)),
        pl.BlockSpec((d, d), lambda i: (0, 0)),
        pl.BlockSpec((1, d), lambda i: (0, 0)),
    ]
    args = [h, st, gf.reshape(1, d), bf.reshape(1, d), wf, bfc.reshape(1, d)]
    if x_add is not None:
        in_specs.append(pl.BlockSpec((blk, d), lambda i: (i, 0)))
        args.append(x_add)
    return pl.pallas_call(
        _v_body_mk(n, x_add is not None),
        grid=(grid,),
        out_shape=jax.ShapeDtypeStruct((n, d), _F32),
        in_specs=in_specs,
        out_specs=pl.BlockSpec((blk, d), lambda i: (i, 0)),
        interpret=interpret,
    )(*args)


def kernel(x, edge_index, edge_weights, edge_times, node_time,
           delta1, ge1, be1, Ws1, bs1, Wh1, bh1, gf1, bf1, Wf1, bfc1,
           delta2, ge2, be2, Ws2, bs2, Wh2, bh2, gf2, bf2, Wf2, bfc2):
    n, d = x.shape
    e_count = edge_index.shape[1]
    src = edge_index[0]
    dst = edge_index[1]

    info = plsc.get_sparse_core_info()
    nc, ns = info.num_cores, info.num_subcores
    nw = nc * ns
    per = e_count // nw
    assert per * nw == e_count
    k = 16
    for cand in (128, 112, 96, 80, 64, 48, 32, 16):
        if per % cand == 0:
            k = cand
            break
    ch = per // k
    sub = ch
    for cand in (25, 20, 16, 10, 8, 5, 4, 2, 1):
        if ch % cand == 0:
            sub = cand
            break
    sgs = ch // sub
    shp = (nw, sgs, sub, k)

    w1, ex1, w2, ex2 = _edge_weights(edge_weights, edge_times,
                                     ge1, be1, delta1, ge2, be2, delta2)

    src_r = src.reshape(shp)
    dst_r = dst.reshape(shp)

    raw1, den1 = _edge_call(src_r, dst_r, w1.reshape(shp),
                            ex1.reshape(shp), x, nc, ns)
    h1, st1 = _u_call(x, raw1, den1.reshape(nc, n).T, Ws1, bs1, Wh1, bh1)
    x2 = _v_call(h1, st1, gf1, bf1, Wf1, bfc1, x_add=x)

    raw2, den2 = _edge_call(src_r, dst_r, w2.reshape(shp),
                            ex2.reshape(shp), x2, nc, ns)
    h2, st2 = _u_call(x2, raw2, den2.reshape(nc, n).T, Ws2, bs2, Wh2, bh2)
    out = _v_call(h2, st2, gf2, bf2, Wf2, bfc2, x_add=None)
    return out
